# Initial kernel scaffold; baseline (speedup 1.0000x reference)
import functools

import jax
import jax.numpy as jnp
from jax import lax
from jax.experimental import pallas as pl
from jax.experimental.pallas import tpu as pltpu
from jax.experimental.pallas import tpu_sc as plsc

N_NODES = 10000
N_EDGES = 160000
D_FEAT = 256
DH = D_FEAT // 2

NC, NS = 2, 16
EDGES_PER_TILE = N_EDGES // NS
BATCH = 80
NBATCH = EDGES_PER_TILE // BATCH
ROWS_PER_TILE = N_NODES // NS

_MESH = plsc.VectorSubcoreMesh(
    core_axis_name="c", subcore_axis_name="s", num_cores=NC, num_subcores=NS
)


@functools.partial(
    pl.kernel,
    out_type=jax.ShapeDtypeStruct((2, N_NODES), jnp.float32),
    mesh=_MESH,
    scratch_types=[
        pltpu.VMEM((BATCH,), jnp.int32),
        pltpu.VMEM((BATCH,), jnp.float32),
        pltpu.VMEM_SHARED((N_NODES,), jnp.float32),
    ],
)
def _deg_kernel(src_hbm, dst_hbm, z_hbm, deg_hbm, idx_v, ones_v, deg_sh):
    c = lax.axis_index("c")
    s = lax.axis_index("s")
    for i in range(BATCH // 16):
        ones_v[pl.ds(i * 16, 16)] = jnp.ones((16,), jnp.float32)

    @pl.when(s == 0)
    def _():
        pltpu.sync_copy(z_hbm.at[c], deg_sh)

    plsc.subcore_barrier()
    base = s * EDGES_PER_TILE

    def body(b, carry):
        off = base + b * BATCH

        @pl.when(c == 0)
        def _():
            pltpu.sync_copy(src_hbm.at[pl.ds(off, BATCH)], idx_v)

        @pl.when(c == 1)
        def _():
            pltpu.sync_copy(dst_hbm.at[pl.ds(off, BATCH)], idx_v)

        pltpu.sync_copy(ones_v, deg_sh.at[idx_v], add=True)
        return carry

    lax.fori_loop(0, NBATCH, body, 0)
    plsc.subcore_barrier()

    @pl.when(s == 0)
    def _():
        pltpu.sync_copy(deg_sh, deg_hbm.at[c])


@functools.partial(
    pl.kernel,
    out_type=jax.ShapeDtypeStruct((2, N_NODES, DH), jnp.float32),
    mesh=_MESH,
    scratch_types=[
        pltpu.VMEM((BATCH,), jnp.int32),
        pltpu.VMEM((BATCH,), jnp.int32),
        pltpu.VMEM((BATCH, DH), jnp.float32),
        pltpu.VMEM_SHARED((N_NODES, DH), jnp.float32),
        pltpu.SemaphoreType.DMA,
    ],
)
def _agg_kernel(xs0_hbm, xs1_hbm, src_hbm, dst_hbm, z_hbm, out_hbm,
                si_v, di_v, rows_v, acc_sh, sem):
    c = lax.axis_index("c")
    s = lax.axis_index("s")
    r0 = s * ROWS_PER_TILE
    pltpu.sync_copy(z_hbm.at[c, pl.ds(r0, ROWS_PER_TILE)],
                    acc_sh.at[pl.ds(r0, ROWS_PER_TILE)])
    plsc.subcore_barrier()
    base = s * EDGES_PER_TILE

    def body(b, carry):
        off = base + b * BATCH
        pltpu.sync_copy(src_hbm.at[pl.ds(off, BATCH)], si_v)
        pltpu.sync_copy(dst_hbm.at[pl.ds(off, BATCH)], di_v)

        @pl.when(c == 0)
        def _():
            pltpu.async_copy(xs0_hbm.at[si_v], rows_v, sem).wait()

        @pl.when(c == 1)
        def _():
            pltpu.async_copy(xs1_hbm.at[si_v], rows_v, sem).wait()

        pltpu.sync_copy(rows_v, acc_sh.at[di_v], add=True)
        return carry

    lax.fori_loop(0, NBATCH, body, 0)
    plsc.subcore_barrier()
    pltpu.sync_copy(acc_sh.at[pl.ds(r0, ROWS_PER_TILE)],
                    out_hbm.at[c, pl.ds(r0, ROWS_PER_TILE)])


_ROWS_BLK = 1000


def _scale_body(feat_ref, deg_ref, o_ref):
    d = jnp.maximum(deg_ref[...], 1.0)
    o_ref[0] = feat_ref[...] * lax.rsqrt(d)


_scale_call = pl.pallas_call(
    _scale_body,
    out_shape=jax.ShapeDtypeStruct((2, N_NODES, DH), jnp.float32),
    grid=(N_NODES // _ROWS_BLK, 2),
    in_specs=[
        pl.BlockSpec((_ROWS_BLK, DH), lambda i, h: (i, h)),
        pl.BlockSpec((_ROWS_BLK, 1), lambda i, h: (i, 0)),
    ],
    out_specs=pl.BlockSpec((1, _ROWS_BLK, DH), lambda i, h: (h, i, 0)),
)


def _fin_body(rst_ref, deg_ref, o_ref):
    d = jnp.maximum(deg_ref[...], 1.0)
    o_ref[...] = rst_ref[0] * lax.rsqrt(d)


_fin_call = pl.pallas_call(
    _fin_body,
    out_shape=jax.ShapeDtypeStruct((N_NODES, D_FEAT), jnp.float32),
    grid=(N_NODES // _ROWS_BLK, 2),
    in_specs=[
        pl.BlockSpec((1, _ROWS_BLK, DH), lambda i, h: (h, i, 0)),
        pl.BlockSpec((_ROWS_BLK, 1), lambda i, h: (i, 0)),
    ],
    out_specs=pl.BlockSpec((_ROWS_BLK, DH), lambda i, h: (i, h)),
)


def kernel(feature, edge_index):
    ei = edge_index.astype(jnp.int32)
    src = ei[0]
    dst = ei[1]
    z2 = jnp.zeros((2, N_NODES), jnp.float32)
    degs = _deg_kernel(src, dst, z2)
    out_deg = degs[0].reshape(N_NODES, 1)
    in_deg = degs[1].reshape(N_NODES, 1)
    xs = _scale_call(feature, out_deg)
    zacc = jnp.zeros((2, N_NODES, DH), jnp.float32)
    rst2 = _agg_kernel(xs[0], xs[1], src, dst, zacc)
    return _fin_call(rst2, in_deg)


# speedup vs baseline: 3.3404x; 3.3404x over previous
import functools

import jax
import jax.numpy as jnp
from jax import lax
from jax.experimental import pallas as pl
from jax.experimental.pallas import tpu as pltpu
from jax.experimental.pallas import tpu_sc as plsc

N_NODES = 10000
N_EDGES = 160000
D_FEAT = 256
DH = D_FEAT // 2

NC, NS = 2, 16
EDGES_PER_TILE = N_EDGES // NS
BATCH = 80
NBATCH = EDGES_PER_TILE // BATCH
N_PAD = 10240
ROWS_PER_TILE = N_PAD // NS

_MESH = plsc.VectorSubcoreMesh(
    core_axis_name="c", subcore_axis_name="s", num_cores=NC, num_subcores=NS
)


@functools.partial(
    pl.kernel,
    out_type=jax.ShapeDtypeStruct((2, N_NODES), jnp.float32),
    mesh=_MESH,
    scratch_types=[
        pltpu.VMEM((BATCH,), jnp.int32),
        pltpu.VMEM((BATCH,), jnp.float32),
        pltpu.VMEM_SHARED((N_NODES,), jnp.float32),
    ],
)
def _deg_kernel(src_hbm, dst_hbm, z_hbm, deg_hbm, idx_v, ones_v, deg_sh):
    c = lax.axis_index("c")
    s = lax.axis_index("s")
    for i in range(BATCH // 16):
        ones_v[pl.ds(i * 16, 16)] = jnp.ones((16,), jnp.float32)

    @pl.when(s == 0)
    def _():
        pltpu.sync_copy(z_hbm.at[c], deg_sh)

    plsc.subcore_barrier()
    base = s * EDGES_PER_TILE

    def body(b, carry):
        off = base + b * BATCH

        @pl.when(c == 0)
        def _():
            pltpu.sync_copy(src_hbm.at[pl.ds(off, BATCH)], idx_v)

        @pl.when(c == 1)
        def _():
            pltpu.sync_copy(dst_hbm.at[pl.ds(off, BATCH)], idx_v)

        pltpu.sync_copy(ones_v, deg_sh.at[idx_v], add=True)
        return carry

    lax.fori_loop(0, NBATCH, body, 0)
    plsc.subcore_barrier()

    @pl.when(s == 0)
    def _():
        pltpu.sync_copy(deg_sh, deg_hbm.at[c])


@functools.partial(
    pl.kernel,
    out_type=jax.ShapeDtypeStruct((2, N_PAD, DH), jnp.float32),
    mesh=_MESH,
    scratch_types=[
        pltpu.VMEM((BATCH,), jnp.int32),
        pltpu.VMEM((BATCH,), jnp.int32),
        pltpu.VMEM((BATCH, DH), jnp.float32),
        pltpu.VMEM_SHARED((N_PAD, DH), jnp.float32),
        pltpu.SemaphoreType.DMA,
    ],
)
def _agg_kernel(xs0_hbm, xs1_hbm, src_hbm, dst_hbm, z_hbm, out_hbm,
                si_v, di_v, rows_v, acc_sh, sem):
    c = lax.axis_index("c")
    s = lax.axis_index("s")
    r0 = s * ROWS_PER_TILE
    pltpu.sync_copy(z_hbm.at[c, pl.ds(r0, ROWS_PER_TILE)],
                    acc_sh.at[pl.ds(r0, ROWS_PER_TILE)])
    plsc.subcore_barrier()
    base = s * EDGES_PER_TILE

    def body(b, carry):
        off = base + b * BATCH
        pltpu.sync_copy(src_hbm.at[pl.ds(off, BATCH)], si_v)
        pltpu.sync_copy(dst_hbm.at[pl.ds(off, BATCH)], di_v)

        @pl.when(c == 0)
        def _():
            pltpu.async_copy(xs0_hbm.at[si_v], rows_v, sem).wait()

        @pl.when(c == 1)
        def _():
            pltpu.async_copy(xs1_hbm.at[si_v], rows_v, sem).wait()

        pltpu.sync_copy(rows_v, acc_sh.at[di_v], add=True)
        return carry

    lax.fori_loop(0, NBATCH, body, 0)
    plsc.subcore_barrier()
    pltpu.sync_copy(acc_sh.at[pl.ds(r0, ROWS_PER_TILE)],
                    out_hbm.at[c, pl.ds(r0, ROWS_PER_TILE)])


_ROWS_BLK = 1000


def _scale_body(feat_ref, deg_ref, o_ref):
    d = jnp.maximum(deg_ref[...], 1.0)
    o_ref[0] = feat_ref[...] * lax.rsqrt(d)


_scale_call = pl.pallas_call(
    _scale_body,
    out_shape=jax.ShapeDtypeStruct((2, N_NODES, DH), jnp.float32),
    grid=(N_NODES // _ROWS_BLK, 2),
    in_specs=[
        pl.BlockSpec((_ROWS_BLK, DH), lambda i, h: (i, h)),
        pl.BlockSpec((_ROWS_BLK, 1), lambda i, h: (i, 0)),
    ],
    out_specs=pl.BlockSpec((1, _ROWS_BLK, DH), lambda i, h: (h, i, 0)),
)


def _fin_body(rst_ref, deg_ref, o_ref):
    d = jnp.maximum(deg_ref[...], 1.0)
    o_ref[...] = rst_ref[0] * lax.rsqrt(d)


_fin_call = pl.pallas_call(
    _fin_body,
    out_shape=jax.ShapeDtypeStruct((N_NODES, D_FEAT), jnp.float32),
    grid=(N_NODES // _ROWS_BLK, 2),
    in_specs=[
        pl.BlockSpec((1, _ROWS_BLK, DH), lambda i, h: (h, i, 0)),
        pl.BlockSpec((_ROWS_BLK, 1), lambda i, h: (i, 0)),
    ],
    out_specs=pl.BlockSpec((_ROWS_BLK, DH), lambda i, h: (i, h)),
)


def kernel(feature, edge_index):
    ei = edge_index.astype(jnp.int32)
    src = ei[0]
    dst = ei[1]
    z2 = jnp.zeros((2, N_NODES), jnp.float32)
    degs = _deg_kernel(src, dst, z2)
    out_deg = degs[0].reshape(N_NODES, 1)
    in_deg = degs[1].reshape(N_NODES, 1)
    xs = _scale_call(feature, out_deg)
    zacc = jnp.zeros((2, N_PAD, DH), jnp.float32)
    rst2 = _agg_kernel(xs[0], xs[1], src, dst, zacc)
    return _fin_call(rst2, in_deg)


# speedup vs baseline: 5.8026x; 1.7371x over previous
import functools

import jax
import jax.numpy as jnp
from jax import lax
from jax.experimental import pallas as pl
from jax.experimental.pallas import tpu as pltpu
from jax.experimental.pallas import tpu_sc as plsc

N_NODES = 10000
N_EDGES = 160000
D_FEAT = 256
DH = D_FEAT // 2

NC, NS = 2, 16
EDGES_PER_TILE = N_EDGES // NS
BATCH = 80
NBATCH = EDGES_PER_TILE // BATCH
N_PAD = 10240
ROWS_PER_TILE = N_PAD // NS

_MESH = plsc.VectorSubcoreMesh(
    core_axis_name="c", subcore_axis_name="s", num_cores=NC, num_subcores=NS
)


@functools.partial(
    pl.kernel,
    out_type=jax.ShapeDtypeStruct((2, N_NODES), jnp.float32),
    mesh=_MESH,
    scratch_types=[
        pltpu.VMEM((BATCH,), jnp.int32),
        pltpu.VMEM((BATCH,), jnp.float32),
        pltpu.VMEM_SHARED((N_NODES,), jnp.float32),
    ],
)
def _deg_kernel(src_hbm, dst_hbm, z_hbm, deg_hbm, idx_v, ones_v, deg_sh):
    c = lax.axis_index("c")
    s = lax.axis_index("s")
    for i in range(BATCH // 16):
        ones_v[pl.ds(i * 16, 16)] = jnp.ones((16,), jnp.float32)

    @pl.when(s == 0)
    def _():
        pltpu.sync_copy(z_hbm.at[c], deg_sh)

    plsc.subcore_barrier()
    base = s * EDGES_PER_TILE

    def body(b, carry):
        off = base + b * BATCH

        @pl.when(c == 0)
        def _():
            pltpu.sync_copy(src_hbm.at[pl.ds(off, BATCH)], idx_v)

        @pl.when(c == 1)
        def _():
            pltpu.sync_copy(dst_hbm.at[pl.ds(off, BATCH)], idx_v)

        pltpu.sync_copy(ones_v, deg_sh.at[idx_v], add=True)
        return carry

    lax.fori_loop(0, NBATCH, body, 0)
    plsc.subcore_barrier()

    @pl.when(s == 0)
    def _():
        pltpu.sync_copy(deg_sh, deg_hbm.at[c])


@functools.partial(
    pl.kernel,
    out_type=jax.ShapeDtypeStruct((2, N_PAD, DH), jnp.float32),
    mesh=_MESH,
    scratch_types=[
        pltpu.VMEM((4, BATCH), jnp.int32),
        pltpu.VMEM((4, BATCH), jnp.int32),
        pltpu.VMEM((2, BATCH, DH), jnp.float32),
        pltpu.VMEM_SHARED((N_PAD, DH), jnp.float32),
        pltpu.SemaphoreType.DMA((4,)),
        pltpu.SemaphoreType.DMA((2,)),
    ],
)
def _agg_kernel(xs0_hbm, xs1_hbm, src_hbm, dst_hbm, z_hbm, out_hbm,
                si_v, di_v, rows_v, acc_sh, isem, gsem):
    c = lax.axis_index("c")
    s = lax.axis_index("s")
    r0 = s * ROWS_PER_TILE
    base = s * EDGES_PER_TILE

    def start_idx(b):
        sl = b % 4
        off = base + b * BATCH
        pltpu.async_copy(src_hbm.at[pl.ds(off, BATCH)], si_v.at[sl],
                         isem.at[sl])
        pltpu.async_copy(dst_hbm.at[pl.ds(off, BATCH)], di_v.at[sl],
                         isem.at[sl])

    def wait_idx(b):
        sl = b % 4
        off = base + b * BATCH
        pltpu.make_async_copy(src_hbm.at[pl.ds(off, BATCH)], si_v.at[sl],
                              isem.at[sl]).wait()
        pltpu.make_async_copy(dst_hbm.at[pl.ds(off, BATCH)], di_v.at[sl],
                              isem.at[sl]).wait()

    def start_gather(b):
        @pl.when(c == 0)
        def _():
            pltpu.async_copy(xs0_hbm.at[si_v.at[b % 4]], rows_v.at[b % 2],
                             gsem.at[b % 2])

        @pl.when(c == 1)
        def _():
            pltpu.async_copy(xs1_hbm.at[si_v.at[b % 4]], rows_v.at[b % 2],
                             gsem.at[b % 2])

    def wait_gather(b):
        pltpu.make_async_copy(xs0_hbm.at[si_v.at[b % 4]], rows_v.at[b % 2],
                              gsem.at[b % 2]).wait()

    start_idx(0)
    start_idx(1)
    pltpu.sync_copy(z_hbm.at[c, pl.ds(r0, ROWS_PER_TILE)],
                    acc_sh.at[pl.ds(r0, ROWS_PER_TILE)])
    wait_idx(0)
    start_gather(0)
    plsc.subcore_barrier()

    def body(b, carry):
        @pl.when(b + 2 < NBATCH)
        def _():
            start_idx(b + 2)

        wait_idx(b + 1)
        start_gather(b + 1)
        wait_gather(b)
        pltpu.sync_copy(rows_v.at[b % 2], acc_sh.at[di_v.at[b % 4]], add=True)
        return carry

    lax.fori_loop(0, NBATCH - 1, body, 0)
    wait_gather(NBATCH - 1)
    pltpu.sync_copy(rows_v.at[(NBATCH - 1) % 2],
                    acc_sh.at[di_v.at[(NBATCH - 1) % 4]], add=True)
    plsc.subcore_barrier()
    pltpu.sync_copy(acc_sh.at[pl.ds(r0, ROWS_PER_TILE)],
                    out_hbm.at[c, pl.ds(r0, ROWS_PER_TILE)])


_ROWS_BLK = 1000


def _scale_body(feat_ref, deg_ref, o_ref):
    d = jnp.maximum(deg_ref[...], 1.0)
    o_ref[0] = feat_ref[...] * lax.rsqrt(d)


_scale_call = pl.pallas_call(
    _scale_body,
    out_shape=jax.ShapeDtypeStruct((2, N_NODES, DH), jnp.float32),
    grid=(N_NODES // _ROWS_BLK, 2),
    in_specs=[
        pl.BlockSpec((_ROWS_BLK, DH), lambda i, h: (i, h)),
        pl.BlockSpec((_ROWS_BLK, 1), lambda i, h: (i, 0)),
    ],
    out_specs=pl.BlockSpec((1, _ROWS_BLK, DH), lambda i, h: (h, i, 0)),
)


def _fin_body(rst_ref, deg_ref, o_ref):
    d = jnp.maximum(deg_ref[...], 1.0)
    o_ref[...] = rst_ref[0] * lax.rsqrt(d)


_fin_call = pl.pallas_call(
    _fin_body,
    out_shape=jax.ShapeDtypeStruct((N_NODES, D_FEAT), jnp.float32),
    grid=(N_NODES // _ROWS_BLK, 2),
    in_specs=[
        pl.BlockSpec((1, _ROWS_BLK, DH), lambda i, h: (h, i, 0)),
        pl.BlockSpec((_ROWS_BLK, 1), lambda i, h: (i, 0)),
    ],
    out_specs=pl.BlockSpec((_ROWS_BLK, DH), lambda i, h: (i, h)),
)


def kernel(feature, edge_index):
    ei = edge_index.astype(jnp.int32)
    src = ei[0]
    dst = ei[1]
    z2 = jnp.zeros((2, N_NODES), jnp.float32)
    degs = _deg_kernel(src, dst, z2)
    out_deg = degs[0].reshape(N_NODES, 1)
    in_deg = degs[1].reshape(N_NODES, 1)
    xs = _scale_call(feature, out_deg)
    zacc = jnp.zeros((2, N_PAD, DH), jnp.float32)
    rst2 = _agg_kernel(xs[0], xs[1], src, dst, zacc)
    return _fin_call(rst2, in_deg)


# speedup vs baseline: 7.0488x; 1.2148x over previous
import functools

import jax
import jax.numpy as jnp
from jax import lax
from jax.experimental import pallas as pl
from jax.experimental.pallas import tpu as pltpu
from jax.experimental.pallas import tpu_sc as plsc

N_NODES = 10000
N_EDGES = 160000
D_FEAT = 256
DH = D_FEAT // 2

NC, NS = 2, 16
EDGES_PER_TILE = N_EDGES // NS
BATCH = 80
NBATCH = EDGES_PER_TILE // BATCH
N_PAD = 10240
ROWS_PER_TILE = N_PAD // NS

_MESH = plsc.VectorSubcoreMesh(
    core_axis_name="c", subcore_axis_name="s", num_cores=NC, num_subcores=NS
)


BATCH_D = 40
NBATCH_D = (N_EDGES // (NC * NS)) // BATCH_D


@functools.partial(
    pl.kernel,
    out_type=jax.ShapeDtypeStruct((2, 1, N_PAD), jnp.float32),
    mesh=_MESH,
    scratch_types=[
        pltpu.VMEM((4, BATCH_D), jnp.int32),
        pltpu.VMEM((48,), jnp.float32),
        pltpu.VMEM_SHARED((N_PAD,), jnp.float32),
        pltpu.SemaphoreType.DMA((4,)),
    ],
)
def _odeg_kernel(src_hbm, z_hbm, deg_hbm, idx_v, ones_v, deg_sh, isem):
    c = lax.axis_index("c")
    s = lax.axis_index("s")
    for i in range(3):
        ones_v[pl.ds(i * 16, 16)] = jnp.ones((16,), jnp.float32)
    base = (c * NS + s) * (BATCH_D * NBATCH_D)

    def start_idx(b):
        pltpu.async_copy(src_hbm.at[pl.ds(base + b * BATCH_D, BATCH_D)],
                         idx_v.at[b % 4], isem.at[b % 4])

    def wait_idx(b):
        pltpu.make_async_copy(src_hbm.at[pl.ds(base + b * BATCH_D, BATCH_D)],
                              idx_v.at[b % 4], isem.at[b % 4]).wait()

    start_idx(0)
    start_idx(1)

    @pl.when(s == 0)
    def _():
        pltpu.sync_copy(z_hbm.at[c, 0], deg_sh)

    plsc.subcore_barrier()

    def body(b, carry):
        @pl.when(b + 2 < NBATCH_D)
        def _():
            start_idx(b + 2)

        wait_idx(b)
        pltpu.sync_copy(ones_v.at[pl.ds(0, BATCH_D)], deg_sh.at[idx_v.at[b % 4]],
                        add=True)
        return carry

    lax.fori_loop(0, NBATCH_D, body, 0)
    plsc.subcore_barrier()

    @pl.when(s == 0)
    def _():
        pltpu.sync_copy(deg_sh, deg_hbm.at[c, 0])


@functools.partial(
    pl.kernel,
    out_type=(jax.ShapeDtypeStruct((2, N_PAD, DH), jnp.float32),
              jax.ShapeDtypeStruct((2, 1, N_PAD), jnp.float32)),
    mesh=_MESH,
    scratch_types=[
        pltpu.VMEM((4, BATCH), jnp.int32),
        pltpu.VMEM((4, BATCH), jnp.int32),
        pltpu.VMEM((2, BATCH, DH), jnp.float32),
        pltpu.VMEM((BATCH,), jnp.float32),
        pltpu.VMEM_SHARED((N_PAD, DH), jnp.float32),
        pltpu.VMEM_SHARED((N_PAD,), jnp.float32),
        pltpu.SemaphoreType.DMA((4,)),
        pltpu.SemaphoreType.DMA((2,)),
        pltpu.SemaphoreType.DMA((4,)),
    ],
)
def _agg_kernel(xs0_hbm, xs1_hbm, src_hbm, dst_hbm, z_hbm, zrow_hbm,
                out_hbm, indeg_hbm,
                si_v, di_v, rows_v, ones_v, acc_sh, indeg_sh, isem, gsem, osem):
    c = lax.axis_index("c")
    s = lax.axis_index("s")
    r0 = s * ROWS_PER_TILE
    base = s * EDGES_PER_TILE
    for i in range(BATCH // 16):
        ones_v[pl.ds(i * 16, 16)] = jnp.ones((16,), jnp.float32)

    def start_idx(b):
        sl = b % 4
        off = base + b * BATCH
        pltpu.async_copy(src_hbm.at[pl.ds(off, BATCH)], si_v.at[sl],
                         isem.at[sl])
        pltpu.async_copy(dst_hbm.at[pl.ds(off, BATCH)], di_v.at[sl],
                         isem.at[sl])

    def wait_idx(b):
        sl = b % 4
        off = base + b * BATCH
        pltpu.make_async_copy(src_hbm.at[pl.ds(off, BATCH)], si_v.at[sl],
                              isem.at[sl]).wait()
        pltpu.make_async_copy(dst_hbm.at[pl.ds(off, BATCH)], di_v.at[sl],
                              isem.at[sl]).wait()

    def start_gather(b):
        @pl.when(c == 0)
        def _():
            pltpu.async_copy(xs0_hbm.at[si_v.at[b % 4]], rows_v.at[b % 2],
                             gsem.at[b % 2])

        @pl.when(c == 1)
        def _():
            pltpu.async_copy(xs1_hbm.at[si_v.at[b % 4]], rows_v.at[b % 2],
                             gsem.at[b % 2])

    def wait_gather(b):
        pltpu.make_async_copy(xs0_hbm.at[si_v.at[b % 4]], rows_v.at[b % 2],
                              gsem.at[b % 2]).wait()

    def start_ones(b):
        pltpu.async_copy(ones_v, indeg_sh.at[di_v.at[b % 4]], osem.at[b % 4],
                         add=True)

    def wait_ones(b):
        pltpu.make_async_copy(ones_v, indeg_sh.at[di_v.at[b % 4]],
                              osem.at[b % 4]).wait()

    start_idx(0)
    start_idx(1)
    pltpu.sync_copy(z_hbm.at[c, pl.ds(r0, ROWS_PER_TILE)],
                    acc_sh.at[pl.ds(r0, ROWS_PER_TILE)])

    @pl.when(s == 0)
    def _():
        pltpu.sync_copy(zrow_hbm.at[c, 0], indeg_sh)

    wait_idx(0)
    start_gather(0)
    plsc.subcore_barrier()

    def body(b, carry):
        @pl.when(b >= 2)
        def _():
            wait_ones(b - 2)

        @pl.when(b + 2 < NBATCH)
        def _():
            start_idx(b + 2)

        wait_idx(b + 1)
        start_gather(b + 1)
        wait_gather(b)
        pltpu.sync_copy(rows_v.at[b % 2], acc_sh.at[di_v.at[b % 4]], add=True)
        start_ones(b)
        return carry

    lax.fori_loop(0, NBATCH - 1, body, 0)
    wait_ones(NBATCH - 3)
    wait_ones(NBATCH - 2)
    wait_gather(NBATCH - 1)
    pltpu.sync_copy(rows_v.at[(NBATCH - 1) % 2],
                    acc_sh.at[di_v.at[(NBATCH - 1) % 4]], add=True)
    pltpu.sync_copy(ones_v, indeg_sh.at[di_v.at[(NBATCH - 1) % 4]], add=True)
    plsc.subcore_barrier()
    pltpu.sync_copy(acc_sh.at[pl.ds(r0, ROWS_PER_TILE)],
                    out_hbm.at[c, pl.ds(r0, ROWS_PER_TILE)])

    @pl.when(s == 0)
    def _():
        pltpu.sync_copy(indeg_sh, indeg_hbm.at[c, 0])


_ROWS_BLK = 1000


def _scale_body(feat_ref, degp_ref, o0_ref, o1_ref):
    d = jnp.maximum(degp_ref[0] + degp_ref[1], 1.0)
    n = lax.rsqrt(d)
    o0_ref[...] = feat_ref[:, :DH] * n
    o1_ref[...] = feat_ref[:, DH:] * n


_scale_call = pl.pallas_call(
    _scale_body,
    out_shape=(jax.ShapeDtypeStruct((N_NODES, DH), jnp.float32),
               jax.ShapeDtypeStruct((N_NODES, DH), jnp.float32)),
    grid=(N_NODES // _ROWS_BLK,),
    in_specs=[
        pl.BlockSpec((_ROWS_BLK, D_FEAT), lambda i: (i, 0)),
        pl.BlockSpec((2, _ROWS_BLK, 1), lambda i: (0, i, 0)),
    ],
    out_specs=(pl.BlockSpec((_ROWS_BLK, DH), lambda i: (i, 0)),
               pl.BlockSpec((_ROWS_BLK, DH), lambda i: (i, 0))),
)


def _fin_body(rst_ref, deg_ref, o_ref):
    d = jnp.maximum(deg_ref[...], 1.0)
    o_ref[...] = rst_ref[0] * lax.rsqrt(d)


_fin_call = pl.pallas_call(
    _fin_body,
    out_shape=jax.ShapeDtypeStruct((N_NODES, D_FEAT), jnp.float32),
    grid=(N_NODES // _ROWS_BLK, 2),
    in_specs=[
        pl.BlockSpec((1, _ROWS_BLK, DH), lambda i, h: (h, i, 0)),
        pl.BlockSpec((_ROWS_BLK, 1), lambda i, h: (i, 0)),
    ],
    out_specs=pl.BlockSpec((_ROWS_BLK, DH), lambda i, h: (i, h)),
)


def kernel(feature, edge_index):
    ei = edge_index.astype(jnp.int32)
    src = ei[0]
    dst = ei[1]
    zrow = jnp.zeros((2, 1, N_PAD), jnp.float32)
    degp = _odeg_kernel(src, zrow)
    xs0, xs1 = _scale_call(feature, degp[:, 0, :N_NODES].reshape(2, N_NODES, 1))
    zacc = jnp.zeros((2, N_PAD, DH), jnp.float32)
    rst2, indeg2 = _agg_kernel(xs0, xs1, src, dst, zacc, zrow)
    in_deg = indeg2[0, 0, :N_NODES].reshape(N_NODES, 1)
    return _fin_call(rst2, in_deg)


# speedup vs baseline: 7.6502x; 1.0853x over previous
import functools

import jax
import jax.numpy as jnp
from jax import lax
from jax.experimental import pallas as pl
from jax.experimental.pallas import tpu as pltpu
from jax.experimental.pallas import tpu_sc as plsc

N_NODES = 10000
N_EDGES = 160000
D_FEAT = 256
DH = D_FEAT // 2

NC, NS = 2, 16
EDGES_PER_TILE = N_EDGES // NS
BATCH = 80
NBATCH = EDGES_PER_TILE // BATCH
N_PAD = 10240
ROWS_PER_TILE = N_PAD // NS

_MESH = plsc.VectorSubcoreMesh(
    core_axis_name="c", subcore_axis_name="s", num_cores=NC, num_subcores=NS
)


BATCH_D = 40
NBATCH_D = (N_EDGES // (NC * NS)) // BATCH_D


@functools.partial(
    pl.kernel,
    out_type=jax.ShapeDtypeStruct((2, 1, N_PAD), jnp.float32),
    mesh=_MESH,
    scratch_types=[
        pltpu.VMEM((4, BATCH_D), jnp.int32),
        pltpu.VMEM((48,), jnp.float32),
        pltpu.VMEM_SHARED((N_PAD,), jnp.float32),
        pltpu.SemaphoreType.DMA((4,)),
    ],
)
def _odeg_kernel(src_hbm, z_hbm, deg_hbm, idx_v, ones_v, deg_sh, isem):
    c = lax.axis_index("c")
    s = lax.axis_index("s")
    for i in range(3):
        ones_v[pl.ds(i * 16, 16)] = jnp.ones((16,), jnp.float32)
    base = (c * NS + s) * (BATCH_D * NBATCH_D)

    def start_idx(b):
        pltpu.async_copy(src_hbm.at[pl.ds(base + b * BATCH_D, BATCH_D)],
                         idx_v.at[b % 4], isem.at[b % 4])

    def wait_idx(b):
        pltpu.make_async_copy(src_hbm.at[pl.ds(base + b * BATCH_D, BATCH_D)],
                              idx_v.at[b % 4], isem.at[b % 4]).wait()

    start_idx(0)
    start_idx(1)

    @pl.when(s == 0)
    def _():
        pltpu.sync_copy(z_hbm.at[c, 0], deg_sh)

    plsc.subcore_barrier()

    def body(b, carry):
        @pl.when(b + 2 < NBATCH_D)
        def _():
            start_idx(b + 2)

        wait_idx(b)
        pltpu.sync_copy(ones_v.at[pl.ds(0, BATCH_D)], deg_sh.at[idx_v.at[b % 4]],
                        add=True)
        return carry

    lax.fori_loop(0, NBATCH_D, body, 0)
    plsc.subcore_barrier()

    @pl.when(s == 0)
    def _():
        pltpu.sync_copy(deg_sh, deg_hbm.at[c, 0])


@functools.partial(
    pl.kernel,
    out_type=(jax.ShapeDtypeStruct((2, N_PAD, DH), jnp.float32),
              jax.ShapeDtypeStruct((2, 1, N_PAD), jnp.float32)),
    mesh=_MESH,
    scratch_types=[
        pltpu.VMEM((4, BATCH), jnp.int32),
        pltpu.VMEM((4, BATCH), jnp.int32),
        pltpu.VMEM((3, BATCH, DH), jnp.float32),
        pltpu.VMEM((BATCH,), jnp.float32),
        pltpu.VMEM_SHARED((N_PAD, DH), jnp.float32),
        pltpu.VMEM_SHARED((N_PAD,), jnp.float32),
        pltpu.SemaphoreType.DMA((4,)),
        pltpu.SemaphoreType.DMA((3,)),
        pltpu.SemaphoreType.DMA((4,)),
        pltpu.SemaphoreType.DMA((3,)),
    ],
)
def _agg_kernel(xs0_hbm, xs1_hbm, src_hbm, dst_hbm, z_hbm, zrow_hbm,
                out_hbm, indeg_hbm,
                si_v, di_v, rows_v, ones_v, acc_sh, indeg_sh,
                isem, gsem, osem, ssem):
    c = lax.axis_index("c")
    s = lax.axis_index("s")
    r0 = s * ROWS_PER_TILE
    base = s * EDGES_PER_TILE
    for i in range(BATCH // 16):
        ones_v[pl.ds(i * 16, 16)] = jnp.ones((16,), jnp.float32)

    def start_idx(b):
        sl = b % 4
        off = base + b * BATCH
        pltpu.async_copy(src_hbm.at[pl.ds(off, BATCH)], si_v.at[sl],
                         isem.at[sl])
        pltpu.async_copy(dst_hbm.at[pl.ds(off, BATCH)], di_v.at[sl],
                         isem.at[sl])

    def wait_idx(b):
        sl = b % 4
        off = base + b * BATCH
        pltpu.make_async_copy(src_hbm.at[pl.ds(off, BATCH)], si_v.at[sl],
                              isem.at[sl]).wait()
        pltpu.make_async_copy(dst_hbm.at[pl.ds(off, BATCH)], di_v.at[sl],
                              isem.at[sl]).wait()

    def start_gather(b):
        @pl.when(c == 0)
        def _():
            pltpu.async_copy(xs0_hbm.at[si_v.at[b % 4]], rows_v.at[b % 3],
                             gsem.at[b % 3])

        @pl.when(c == 1)
        def _():
            pltpu.async_copy(xs1_hbm.at[si_v.at[b % 4]], rows_v.at[b % 3],
                             gsem.at[b % 3])

    def wait_gather(b):
        pltpu.make_async_copy(xs0_hbm.at[si_v.at[b % 4]], rows_v.at[b % 3],
                              gsem.at[b % 3]).wait()

    def start_scatter(b):
        pltpu.async_copy(rows_v.at[b % 3], acc_sh.at[di_v.at[b % 4]],
                         ssem.at[b % 3], add=True)

    def wait_scatter(b):
        pltpu.make_async_copy(rows_v.at[b % 3], acc_sh.at[di_v.at[b % 4]],
                              ssem.at[b % 3]).wait()

    def start_ones(b):
        pltpu.async_copy(ones_v, indeg_sh.at[di_v.at[b % 4]], osem.at[b % 4],
                         add=True)

    def wait_ones(b):
        pltpu.make_async_copy(ones_v, indeg_sh.at[di_v.at[b % 4]],
                              osem.at[b % 4]).wait()

    start_idx(0)
    start_idx(1)
    pltpu.sync_copy(z_hbm.at[c, pl.ds(r0, ROWS_PER_TILE)],
                    acc_sh.at[pl.ds(r0, ROWS_PER_TILE)])

    @pl.when(s == 0)
    def _():
        pltpu.sync_copy(zrow_hbm.at[c, 0], indeg_sh)

    wait_idx(0)
    start_gather(0)
    plsc.subcore_barrier()

    def body(b, carry):
        @pl.when(b >= 2)
        def _():
            wait_ones(b - 2)
            wait_scatter(b - 2)

        @pl.when(b + 2 < NBATCH)
        def _():
            start_idx(b + 2)

        @pl.when(b + 1 < NBATCH)
        def _():
            wait_idx(b + 1)
            start_gather(b + 1)

        wait_gather(b)
        start_scatter(b)
        start_ones(b)
        return carry

    lax.fori_loop(0, NBATCH, body, 0)
    wait_scatter(NBATCH - 2)
    wait_scatter(NBATCH - 1)
    wait_ones(NBATCH - 2)
    wait_ones(NBATCH - 1)
    plsc.subcore_barrier()
    pltpu.sync_copy(acc_sh.at[pl.ds(r0, ROWS_PER_TILE)],
                    out_hbm.at[c, pl.ds(r0, ROWS_PER_TILE)])

    @pl.when(s == 0)
    def _():
        pltpu.sync_copy(indeg_sh, indeg_hbm.at[c, 0])


_ROWS_BLK = 1000


def _scale_body(feat_ref, degp_ref, o0_ref, o1_ref):
    d = jnp.maximum(degp_ref[0] + degp_ref[1], 1.0)
    n = lax.rsqrt(d)
    o0_ref[...] = feat_ref[:, :DH] * n
    o1_ref[...] = feat_ref[:, DH:] * n


_scale_call = pl.pallas_call(
    _scale_body,
    out_shape=(jax.ShapeDtypeStruct((N_NODES, DH), jnp.float32),
               jax.ShapeDtypeStruct((N_NODES, DH), jnp.float32)),
    grid=(N_NODES // _ROWS_BLK,),
    in_specs=[
        pl.BlockSpec((_ROWS_BLK, D_FEAT), lambda i: (i, 0)),
        pl.BlockSpec((2, _ROWS_BLK, 1), lambda i: (0, i, 0)),
    ],
    out_specs=(pl.BlockSpec((_ROWS_BLK, DH), lambda i: (i, 0)),
               pl.BlockSpec((_ROWS_BLK, DH), lambda i: (i, 0))),
)


def _fin_body(rst_ref, deg_ref, o_ref):
    d = jnp.maximum(deg_ref[...], 1.0)
    o_ref[...] = rst_ref[0] * lax.rsqrt(d)


_fin_call = pl.pallas_call(
    _fin_body,
    out_shape=jax.ShapeDtypeStruct((N_NODES, D_FEAT), jnp.float32),
    grid=(N_NODES // _ROWS_BLK, 2),
    in_specs=[
        pl.BlockSpec((1, _ROWS_BLK, DH), lambda i, h: (h, i, 0)),
        pl.BlockSpec((_ROWS_BLK, 1), lambda i, h: (i, 0)),
    ],
    out_specs=pl.BlockSpec((_ROWS_BLK, DH), lambda i, h: (i, h)),
)


def kernel(feature, edge_index):
    ei = edge_index.astype(jnp.int32)
    src = ei[0]
    dst = ei[1]
    zrow = jnp.zeros((2, 1, N_PAD), jnp.float32)
    degp = _odeg_kernel(src, zrow)
    xs0, xs1 = _scale_call(feature, degp[:, 0, :N_NODES].reshape(2, N_NODES, 1))
    zacc = jnp.zeros((2, N_PAD, DH), jnp.float32)
    rst2, indeg2 = _agg_kernel(xs0, xs1, src, dst, zacc, zrow)
    in_deg = indeg2[0, 0, :N_NODES].reshape(N_NODES, 1)
    return _fin_call(rst2, in_deg)


# speedup vs baseline: 7.7990x; 1.0194x over previous
import functools

import jax
import jax.numpy as jnp
from jax import lax
from jax.experimental import pallas as pl
from jax.experimental.pallas import tpu as pltpu
from jax.experimental.pallas import tpu_sc as plsc

N_NODES = 10000
N_EDGES = 160000
D_FEAT = 256
DH = D_FEAT // 2

NC, NS = 2, 16
EDGES_PER_TILE = N_EDGES // NS
BATCH = 80
NBATCH = EDGES_PER_TILE // BATCH
N_PAD = 10240
ROWS_PER_TILE = N_PAD // NS

_MESH = plsc.VectorSubcoreMesh(
    core_axis_name="c", subcore_axis_name="s", num_cores=NC, num_subcores=NS
)


BATCH_D = 40
NBATCH_D = (N_EDGES // (NC * NS)) // BATCH_D


@functools.partial(
    pl.kernel,
    out_type=jax.ShapeDtypeStruct((2, 1, N_PAD), jnp.float32),
    mesh=_MESH,
    scratch_types=[
        pltpu.VMEM((4, BATCH_D), jnp.int32),
        pltpu.VMEM((48,), jnp.float32),
        pltpu.VMEM_SHARED((N_PAD,), jnp.float32),
        pltpu.SemaphoreType.DMA((4,)),
        pltpu.SemaphoreType.DMA((4,)),
    ],
)
def _odeg_kernel(src_hbm, z_hbm, deg_hbm, idx_v, ones_v, deg_sh, isem, osem):
    c = lax.axis_index("c")
    s = lax.axis_index("s")
    for i in range(3):
        ones_v[pl.ds(i * 16, 16)] = jnp.ones((16,), jnp.float32)
    base = (c * NS + s) * (BATCH_D * NBATCH_D)

    def start_idx(b):
        pltpu.async_copy(src_hbm.at[pl.ds(base + b * BATCH_D, BATCH_D)],
                         idx_v.at[b % 4], isem.at[b % 4])

    def wait_idx(b):
        pltpu.make_async_copy(src_hbm.at[pl.ds(base + b * BATCH_D, BATCH_D)],
                              idx_v.at[b % 4], isem.at[b % 4]).wait()

    start_idx(0)
    start_idx(1)

    @pl.when(s == 0)
    def _():
        pltpu.sync_copy(z_hbm.at[c, 0], deg_sh)

    plsc.subcore_barrier()

    def start_ones(b):
        pltpu.async_copy(ones_v.at[pl.ds(0, BATCH_D)],
                         deg_sh.at[idx_v.at[b % 4]], osem.at[b % 4], add=True)

    def wait_ones(b):
        pltpu.make_async_copy(ones_v.at[pl.ds(0, BATCH_D)],
                              deg_sh.at[idx_v.at[b % 4]], osem.at[b % 4]).wait()

    def body(b, carry):
        @pl.when(b >= 2)
        def _():
            wait_ones(b - 2)

        @pl.when(b + 2 < NBATCH_D)
        def _():
            start_idx(b + 2)

        wait_idx(b)
        start_ones(b)
        return carry

    lax.fori_loop(0, NBATCH_D, body, 0)
    wait_ones(NBATCH_D - 2)
    wait_ones(NBATCH_D - 1)
    plsc.subcore_barrier()

    @pl.when(s == 0)
    def _():
        pltpu.sync_copy(deg_sh, deg_hbm.at[c, 0])


@functools.partial(
    pl.kernel,
    out_type=(jax.ShapeDtypeStruct((2, N_PAD, DH), jnp.float32),
              jax.ShapeDtypeStruct((2, 1, N_PAD), jnp.float32)),
    mesh=_MESH,
    scratch_types=[
        pltpu.VMEM((4, BATCH), jnp.int32),
        pltpu.VMEM((4, BATCH), jnp.int32),
        pltpu.VMEM((3, BATCH, DH), jnp.float32),
        pltpu.VMEM((BATCH,), jnp.float32),
        pltpu.VMEM_SHARED((N_PAD, DH), jnp.float32),
        pltpu.VMEM_SHARED((N_PAD,), jnp.float32),
        pltpu.SemaphoreType.DMA((4,)),
        pltpu.SemaphoreType.DMA((3,)),
        pltpu.SemaphoreType.DMA((4,)),
        pltpu.SemaphoreType.DMA((3,)),
    ],
)
def _agg_kernel(xs0_hbm, xs1_hbm, src_hbm, dst_hbm, z_hbm, zrow_hbm,
                out_hbm, indeg_hbm,
                si_v, di_v, rows_v, ones_v, acc_sh, indeg_sh,
                isem, gsem, osem, ssem):
    c = lax.axis_index("c")
    s = lax.axis_index("s")
    r0 = s * ROWS_PER_TILE
    base = s * EDGES_PER_TILE
    for i in range(BATCH // 16):
        ones_v[pl.ds(i * 16, 16)] = jnp.ones((16,), jnp.float32)

    def start_idx(b):
        sl = b % 4
        off = base + b * BATCH
        pltpu.async_copy(src_hbm.at[pl.ds(off, BATCH)], si_v.at[sl],
                         isem.at[sl])
        pltpu.async_copy(dst_hbm.at[pl.ds(off, BATCH)], di_v.at[sl],
                         isem.at[sl])

    def wait_idx(b):
        sl = b % 4
        off = base + b * BATCH
        pltpu.make_async_copy(src_hbm.at[pl.ds(off, BATCH)], si_v.at[sl],
                              isem.at[sl]).wait()
        pltpu.make_async_copy(dst_hbm.at[pl.ds(off, BATCH)], di_v.at[sl],
                              isem.at[sl]).wait()

    def start_gather(b):
        @pl.when(c == 0)
        def _():
            pltpu.async_copy(xs0_hbm.at[si_v.at[b % 4]], rows_v.at[b % 3],
                             gsem.at[b % 3])

        @pl.when(c == 1)
        def _():
            pltpu.async_copy(xs1_hbm.at[si_v.at[b % 4]], rows_v.at[b % 3],
                             gsem.at[b % 3])

    def wait_gather(b):
        pltpu.make_async_copy(xs0_hbm.at[si_v.at[b % 4]], rows_v.at[b % 3],
                              gsem.at[b % 3]).wait()

    def start_scatter(b):
        pltpu.async_copy(rows_v.at[b % 3], acc_sh.at[di_v.at[b % 4]],
                         ssem.at[b % 3], add=True)

    def wait_scatter(b):
        pltpu.make_async_copy(rows_v.at[b % 3], acc_sh.at[di_v.at[b % 4]],
                              ssem.at[b % 3]).wait()

    def start_ones(b):
        pltpu.async_copy(ones_v, indeg_sh.at[di_v.at[b % 4]], osem.at[b % 4],
                         add=True)

    def wait_ones(b):
        pltpu.make_async_copy(ones_v, indeg_sh.at[di_v.at[b % 4]],
                              osem.at[b % 4]).wait()

    start_idx(0)
    start_idx(1)
    pltpu.sync_copy(z_hbm.at[c, pl.ds(r0, ROWS_PER_TILE)],
                    acc_sh.at[pl.ds(r0, ROWS_PER_TILE)])

    @pl.when(s == 0)
    def _():
        pltpu.sync_copy(zrow_hbm.at[c, 0], indeg_sh)

    wait_idx(0)
    start_gather(0)
    plsc.subcore_barrier()

    def body(b, carry):
        @pl.when(b >= 2)
        def _():
            wait_ones(b - 2)
            wait_scatter(b - 2)

        @pl.when(b + 2 < NBATCH)
        def _():
            start_idx(b + 2)

        @pl.when(b + 1 < NBATCH)
        def _():
            wait_idx(b + 1)
            start_gather(b + 1)

        wait_gather(b)
        start_scatter(b)
        start_ones(b)
        return carry

    lax.fori_loop(0, NBATCH, body, 0)
    wait_scatter(NBATCH - 2)
    wait_scatter(NBATCH - 1)
    wait_ones(NBATCH - 2)
    wait_ones(NBATCH - 1)
    plsc.subcore_barrier()
    pltpu.sync_copy(acc_sh.at[pl.ds(r0, ROWS_PER_TILE)],
                    out_hbm.at[c, pl.ds(r0, ROWS_PER_TILE)])

    @pl.when(s == 0)
    def _():
        pltpu.sync_copy(indeg_sh, indeg_hbm.at[c, 0])


_ROWS_BLK = 1000


def _scale_body(feat_ref, degp_ref, o0_ref, o1_ref, zacc_ref):
    d = jnp.maximum(degp_ref[0] + degp_ref[1], 1.0)
    n = lax.rsqrt(d)
    o0_ref[...] = feat_ref[:, :DH] * n
    o1_ref[...] = feat_ref[:, DH:] * n
    zacc_ref[...] = jnp.zeros_like(zacc_ref)


_scale_call = pl.pallas_call(
    _scale_body,
    out_shape=(jax.ShapeDtypeStruct((N_NODES, DH), jnp.float32),
               jax.ShapeDtypeStruct((N_NODES, DH), jnp.float32),
               jax.ShapeDtypeStruct((2, N_PAD, DH), jnp.float32)),
    grid=(N_NODES // _ROWS_BLK,),
    in_specs=[
        pl.BlockSpec((_ROWS_BLK, D_FEAT), lambda i: (i, 0)),
        pl.BlockSpec((2, _ROWS_BLK, 1), lambda i: (0, i, 0)),
    ],
    out_specs=(pl.BlockSpec((_ROWS_BLK, DH), lambda i: (i, 0)),
               pl.BlockSpec((_ROWS_BLK, DH), lambda i: (i, 0)),
               pl.BlockSpec((2, N_PAD // (N_NODES // _ROWS_BLK), DH),
                            lambda i: (0, i, 0))),
)


def _fin_body(rst_ref, deg_ref, o_ref):
    d = jnp.maximum(deg_ref[...], 1.0)
    o_ref[...] = rst_ref[0] * lax.rsqrt(d)


_fin_call = pl.pallas_call(
    _fin_body,
    out_shape=jax.ShapeDtypeStruct((N_NODES, D_FEAT), jnp.float32),
    grid=(N_NODES // _ROWS_BLK, 2),
    in_specs=[
        pl.BlockSpec((1, _ROWS_BLK, DH), lambda i, h: (h, i, 0)),
        pl.BlockSpec((_ROWS_BLK, 1), lambda i, h: (i, 0)),
    ],
    out_specs=pl.BlockSpec((_ROWS_BLK, DH), lambda i, h: (i, h)),
)


def kernel(feature, edge_index):
    ei = edge_index.astype(jnp.int32)
    src = ei[0]
    dst = ei[1]
    zrow = jnp.zeros((2, 1, N_PAD), jnp.float32)
    degp = _odeg_kernel(src, zrow)
    xs0, xs1, zacc = _scale_call(
        feature, degp[:, 0, :N_NODES].reshape(2, N_NODES, 1))
    rst2, indeg2 = _agg_kernel(xs0, xs1, src, dst, zacc, zrow)
    in_deg = indeg2[0, 0, :N_NODES].reshape(N_NODES, 1)
    return _fin_call(rst2, in_deg)
